# Initial kernel scaffold; baseline (speedup 1.0000x reference)
import jax
import jax.numpy as jnp
from jax.experimental import pallas as pl


def kernel(triple_batch, triple_index, word_embed, p_content, W_ih, W_hh, b_ih, b_hh):
    raise NotImplementedError("write your pallas kernel here")


# speedup vs baseline: 2.6069x; 2.6069x over previous
import functools

import jax
import jax.numpy as jnp
from jax import lax
from jax.experimental import pallas as pl
from jax.experimental.pallas import tpu as pltpu
from jax.experimental.pallas import tpu_sc as plsc

EMBED_D = 128
C_LEN = 100
B3 = 3072
NUM_CORES = 2
NUM_SUBCORES = 16
NW = NUM_CORES * NUM_SUBCORES
BPW = B3 // NW


def _sc_gather_body(ids_hbm, pcontent_hbm, wembed_hbm, emb_hbm,
                    ids_v, tok_v, row_v, sem):
    wid = lax.axis_index("s") * NUM_CORES + lax.axis_index("c")
    base = wid * BPW
    pltpu.sync_copy(ids_hbm.at[pl.ds(base, BPW)], ids_v)
    pltpu.async_copy(pcontent_hbm.at[ids_v], tok_v, sem).wait()

    def body(b, carry):
        idx = tok_v.at[b, pl.ds(0, C_LEN)]
        pltpu.async_copy(wembed_hbm.at[idx], row_v, sem).wait()
        pltpu.sync_copy(row_v, emb_hbm.at[base + b])
        return carry

    lax.fori_loop(0, BPW, body, 0)


def _sc_gather(flat_ids, p_content_pad, word_embed):
    mesh = plsc.VectorSubcoreMesh(
        core_axis_name="c", subcore_axis_name="s",
        num_cores=NUM_CORES, num_subcores=NUM_SUBCORES)
    run = pl.kernel(
        _sc_gather_body,
        out_type=jax.ShapeDtypeStruct((B3, C_LEN, EMBED_D), jnp.float32),
        mesh=mesh,
        scratch_types=[
            pltpu.VMEM((BPW,), jnp.int32),
            pltpu.VMEM((BPW, 128), jnp.int32),
            pltpu.VMEM((C_LEN, EMBED_D), jnp.float32),
            pltpu.SemaphoreType.DMA,
        ],
    )
    return run(flat_ids, p_content_pad, word_embed)


def _lstm_body(emb_ref, wih_ref, whh_ref, bias_ref, out_ref, h_ref, c_ref):
    t = pl.program_id(0)

    @pl.when(t == 0)
    def _init():
        h_ref[...] = jnp.zeros_like(h_ref)
        c_ref[...] = jnp.zeros_like(c_ref)
        out_ref[...] = jnp.zeros_like(out_ref)

    x = emb_ref[...]
    gates = (
        jnp.dot(x, wih_ref[...], preferred_element_type=jnp.float32)
        + jnp.dot(h_ref[...], whh_ref[...], preferred_element_type=jnp.float32)
        + bias_ref[0:1, :]
    )
    i = jax.nn.sigmoid(gates[:, 0:EMBED_D])
    f = jax.nn.sigmoid(gates[:, EMBED_D:2 * EMBED_D])
    g = jnp.tanh(gates[:, 2 * EMBED_D:3 * EMBED_D])
    o = jax.nn.sigmoid(gates[:, 3 * EMBED_D:4 * EMBED_D])
    c = f * c_ref[...] + i * g
    h = o * jnp.tanh(c)
    h_ref[...] = h
    c_ref[...] = c
    out_ref[...] += h

    @pl.when(t == C_LEN - 1)
    def _finish():
        out_ref[...] *= (1.0 / C_LEN)


def _lstm(emb2d, wih_t, whh_t, bias):
    return pl.pallas_call(
        _lstm_body,
        grid=(C_LEN,),
        in_specs=[
            pl.BlockSpec((B3, EMBED_D), lambda t: (0, t)),
            pl.BlockSpec((EMBED_D, 4 * EMBED_D), lambda t: (0, 0)),
            pl.BlockSpec((EMBED_D, 4 * EMBED_D), lambda t: (0, 0)),
            pl.BlockSpec((8, 4 * EMBED_D), lambda t: (0, 0)),
        ],
        out_specs=pl.BlockSpec((B3, EMBED_D), lambda t: (0, 0)),
        out_shape=jax.ShapeDtypeStruct((B3, EMBED_D), jnp.float32),
        scratch_shapes=[
            pltpu.VMEM((B3, EMBED_D), jnp.float32),
            pltpu.VMEM((B3, EMBED_D), jnp.float32),
        ],
    )(emb2d, wih_t, whh_t, bias)


def kernel(triple_batch, triple_index, word_embed, p_content, W_ih, W_hh,
           b_ih, b_hh):
    flat_ids = jnp.transpose(triple_batch.astype(jnp.int32)).reshape(B3)
    p_content_pad = jnp.pad(
        p_content.astype(jnp.int32), ((0, 0), (0, 128 - C_LEN)))
    emb = _sc_gather(flat_ids, p_content_pad, word_embed)

    wih_t = jnp.transpose(W_ih)
    whh_t = jnp.transpose(W_hh)
    bias = jnp.broadcast_to((b_ih + b_hh)[None, :], (8, 4 * EMBED_D))
    out = _lstm(emb.reshape(B3, C_LEN * EMBED_D), wih_t, whh_t, bias)
    return (out[0:1024], out[1024:2048], out[2048:3072])


# speedup vs baseline: 2.8720x; 1.1017x over previous
import functools

import jax
import jax.numpy as jnp
from jax import lax
from jax.experimental import pallas as pl
from jax.experimental.pallas import tpu as pltpu
from jax.experimental.pallas import tpu_sc as plsc

EMBED_D = 128
C_LEN = 100
B3 = 3072
NUM_CORES = 2
NUM_SUBCORES = 16
NW = NUM_CORES * NUM_SUBCORES
BPW = B3 // NW
NB = 4


def _sc_gather_body(ids_hbm, pcontent_hbm, wembed_hbm, emb_hbm,
                    ids_v, tok_v, buf0, buf1, buf2, buf3, sg, ss):
    bufs = (buf0, buf1, buf2, buf3)
    wid = lax.axis_index("s") * NUM_CORES + lax.axis_index("c")
    base = wid * BPW
    pltpu.sync_copy(ids_hbm.at[pl.ds(base, BPW)], ids_v)
    pltpu.async_copy(pcontent_hbm.at[ids_v], tok_v, sg.at[0]).wait()

    def tok_idx(b):
        return tok_v.at[b, pl.ds(0, C_LEN)]

    for j in range(2):
        pltpu.async_copy(wembed_hbm.at[tok_idx(j)], bufs[j], sg.at[j])

    def round_body(g, carry):
        for j in range(NB):
            b = g * NB + j
            pltpu.make_async_copy(
                wembed_hbm.at[tok_idx(b)], bufs[j], sg.at[j]).wait()
            pltpu.async_copy(bufs[j], emb_hbm.at[base + b], ss.at[j])
            f = b + 2
            fs = (j + 2) % NB

            @pl.when(f < BPW)
            def _fire():
                @pl.when(b >= 2)
                def _drain():
                    pltpu.make_async_copy(
                        bufs[fs], emb_hbm.at[base + b], ss.at[fs]).wait()
                pltpu.async_copy(
                    wembed_hbm.at[tok_idx(f)], bufs[fs], sg.at[fs])
        return carry

    lax.fori_loop(0, BPW // NB, round_body, 0)
    for j in range(NB):
        pltpu.make_async_copy(bufs[j], emb_hbm.at[base], ss.at[j]).wait()


def _sc_gather(flat_ids, p_content_pad, word_embed):
    mesh = plsc.VectorSubcoreMesh(
        core_axis_name="c", subcore_axis_name="s",
        num_cores=NUM_CORES, num_subcores=NUM_SUBCORES)
    run = pl.kernel(
        _sc_gather_body,
        out_type=jax.ShapeDtypeStruct((B3, C_LEN, EMBED_D), jnp.float32),
        mesh=mesh,
        scratch_types=[
            pltpu.VMEM((BPW,), jnp.int32),
            pltpu.VMEM((BPW, 128), jnp.int32),
            pltpu.VMEM((C_LEN, EMBED_D), jnp.float32),
            pltpu.VMEM((C_LEN, EMBED_D), jnp.float32),
            pltpu.VMEM((C_LEN, EMBED_D), jnp.float32),
            pltpu.VMEM((C_LEN, EMBED_D), jnp.float32),
            pltpu.SemaphoreType.DMA((NB,)),
            pltpu.SemaphoreType.DMA((NB,)),
        ],
    )
    return run(flat_ids, p_content_pad, word_embed)


def _lstm_body(emb_ref, wih_ref, whh_ref, bias_ref, out_ref, h_ref, c_ref):
    t = pl.program_id(0)

    @pl.when(t == 0)
    def _init():
        h_ref[...] = jnp.zeros_like(h_ref)
        c_ref[...] = jnp.zeros_like(c_ref)
        out_ref[...] = jnp.zeros_like(out_ref)

    x = emb_ref[...].astype(jnp.bfloat16)
    h_bf = h_ref[...].astype(jnp.bfloat16)
    gates = (
        jnp.dot(x, wih_ref[...], preferred_element_type=jnp.float32)
        + jnp.dot(h_bf, whh_ref[...], preferred_element_type=jnp.float32)
        + bias_ref[0:1, :]
    )
    i = jax.nn.sigmoid(gates[:, 0:EMBED_D])
    f = jax.nn.sigmoid(gates[:, EMBED_D:2 * EMBED_D])
    g = jnp.tanh(gates[:, 2 * EMBED_D:3 * EMBED_D])
    o = jax.nn.sigmoid(gates[:, 3 * EMBED_D:4 * EMBED_D])
    c = f * c_ref[...] + i * g
    h = o * jnp.tanh(c)
    h_ref[...] = h
    c_ref[...] = c
    out_ref[...] += h

    @pl.when(t == C_LEN - 1)
    def _finish():
        out_ref[...] *= (1.0 / C_LEN)


def _lstm(emb2d, wih_t, whh_t, bias):
    return pl.pallas_call(
        _lstm_body,
        grid=(C_LEN,),
        in_specs=[
            pl.BlockSpec((B3, EMBED_D), lambda t: (0, t)),
            pl.BlockSpec((EMBED_D, 4 * EMBED_D), lambda t: (0, 0)),
            pl.BlockSpec((EMBED_D, 4 * EMBED_D), lambda t: (0, 0)),
            pl.BlockSpec((8, 4 * EMBED_D), lambda t: (0, 0)),
        ],
        out_specs=pl.BlockSpec((B3, EMBED_D), lambda t: (0, 0)),
        out_shape=jax.ShapeDtypeStruct((B3, EMBED_D), jnp.float32),
        scratch_shapes=[
            pltpu.VMEM((B3, EMBED_D), jnp.float32),
            pltpu.VMEM((B3, EMBED_D), jnp.float32),
        ],
    )(emb2d, wih_t, whh_t, bias)


def kernel(triple_batch, triple_index, word_embed, p_content, W_ih, W_hh,
           b_ih, b_hh):
    flat_ids = jnp.transpose(triple_batch.astype(jnp.int32)).reshape(B3)
    p_content_pad = jnp.pad(
        p_content.astype(jnp.int32), ((0, 0), (0, 128 - C_LEN)))
    emb = _sc_gather(flat_ids, p_content_pad, word_embed)

    wih_t = jnp.transpose(W_ih).astype(jnp.bfloat16)
    whh_t = jnp.transpose(W_hh).astype(jnp.bfloat16)
    bias = jnp.broadcast_to((b_ih + b_hh)[None, :], (8, 4 * EMBED_D))
    out = _lstm(emb.reshape(B3, C_LEN * EMBED_D), wih_t, whh_t, bias)
    return (out[0:1024], out[1024:2048], out[2048:3072])


# speedup vs baseline: 3.0904x; 1.0760x over previous
import functools

import jax
import jax.numpy as jnp
from jax import lax
from jax.experimental import pallas as pl
from jax.experimental.pallas import tpu as pltpu
from jax.experimental.pallas import tpu_sc as plsc

EMBED_D = 128
C_LEN = 100
B3 = 3072
NUM_CORES = 2
NUM_SUBCORES = 16
NW = NUM_CORES * NUM_SUBCORES
BPW = B3 // NW
NB = 4


def _sc_gather_body(ids_hbm, pcontent_hbm, wembed_hbm, emb_hbm,
                    ids_v, tok_v, buf0, buf1, buf2, buf3, sg, ss):
    bufs = (buf0, buf1, buf2, buf3)
    wid = lax.axis_index("s") * NUM_CORES + lax.axis_index("c")
    base = wid * BPW
    pltpu.sync_copy(ids_hbm.at[pl.ds(base, BPW)], ids_v)
    pltpu.async_copy(pcontent_hbm.at[ids_v], tok_v, sg.at[0]).wait()

    def tok_idx(b):
        return tok_v.at[b, pl.ds(0, C_LEN)]

    for j in range(2):
        pltpu.async_copy(wembed_hbm.at[tok_idx(j)], bufs[j], sg.at[j])

    def round_body(g, carry):
        for j in range(NB):
            b = g * NB + j
            pltpu.make_async_copy(
                wembed_hbm.at[tok_idx(b)], bufs[j], sg.at[j]).wait()
            pltpu.async_copy(bufs[j], emb_hbm.at[base + b], ss.at[j])
            f = b + 2
            fs = (j + 2) % NB

            @pl.when(f < BPW)
            def _fire():
                @pl.when(b >= 2)
                def _drain():
                    pltpu.make_async_copy(
                        bufs[fs], emb_hbm.at[base + b], ss.at[fs]).wait()
                pltpu.async_copy(
                    wembed_hbm.at[tok_idx(f)], bufs[fs], sg.at[fs])
        return carry

    lax.fori_loop(0, BPW // NB, round_body, 0)
    for j in range(NB):
        pltpu.make_async_copy(bufs[j], emb_hbm.at[base], ss.at[j]).wait()


def _sc_gather(flat_ids, p_content_pad, word_embed):
    mesh = plsc.VectorSubcoreMesh(
        core_axis_name="c", subcore_axis_name="s",
        num_cores=NUM_CORES, num_subcores=NUM_SUBCORES)
    run = pl.kernel(
        _sc_gather_body,
        out_type=jax.ShapeDtypeStruct((B3, C_LEN, EMBED_D), jnp.float32),
        mesh=mesh,
        scratch_types=[
            pltpu.VMEM((BPW,), jnp.int32),
            pltpu.VMEM((BPW, 128), jnp.int32),
            pltpu.VMEM((C_LEN, EMBED_D), jnp.float32),
            pltpu.VMEM((C_LEN, EMBED_D), jnp.float32),
            pltpu.VMEM((C_LEN, EMBED_D), jnp.float32),
            pltpu.VMEM((C_LEN, EMBED_D), jnp.float32),
            pltpu.SemaphoreType.DMA((NB,)),
            pltpu.SemaphoreType.DMA((NB,)),
        ],
    )
    return run(flat_ids, p_content_pad, word_embed)


T_BLK = 10


def _lstm_body(emb_ref, wih_ref, whh_ref, bias_ref, out_ref, h_ref, c_ref):
    t = pl.program_id(0)

    @pl.when(t == 0)
    def _init():
        h_ref[...] = jnp.zeros_like(h_ref)
        c_ref[...] = jnp.zeros_like(c_ref)
        out_ref[...] = jnp.zeros_like(out_ref)

    acc = out_ref[...]
    h = h_ref[...]
    c = c_ref[...]
    for k in range(T_BLK):
        x = emb_ref[:, k * EMBED_D:(k + 1) * EMBED_D].astype(jnp.bfloat16)
        gates = (
            jnp.dot(x, wih_ref[...], preferred_element_type=jnp.float32)
            + jnp.dot(h.astype(jnp.bfloat16), whh_ref[...],
                      preferred_element_type=jnp.float32)
            + bias_ref[0:1, :]
        )
        i = 0.5 * jnp.tanh(gates[:, 0:EMBED_D]) + 0.5
        f = 0.5 * jnp.tanh(gates[:, EMBED_D:2 * EMBED_D]) + 0.5
        g = jnp.tanh(gates[:, 2 * EMBED_D:3 * EMBED_D])
        o = 0.5 * jnp.tanh(gates[:, 3 * EMBED_D:4 * EMBED_D]) + 0.5
        c = f * c + i * g
        h = o * jnp.tanh(c)
        acc += h
    h_ref[...] = h
    c_ref[...] = c
    out_ref[...] = acc

    @pl.when(t == C_LEN // T_BLK - 1)
    def _finish():
        out_ref[...] = acc * (1.0 / C_LEN)


def _lstm(emb2d, wih_t, whh_t, bias):
    return pl.pallas_call(
        _lstm_body,
        grid=(C_LEN // T_BLK,),
        in_specs=[
            pl.BlockSpec((B3, T_BLK * EMBED_D), lambda t: (0, t)),
            pl.BlockSpec((EMBED_D, 4 * EMBED_D), lambda t: (0, 0)),
            pl.BlockSpec((EMBED_D, 4 * EMBED_D), lambda t: (0, 0)),
            pl.BlockSpec((8, 4 * EMBED_D), lambda t: (0, 0)),
        ],
        out_specs=pl.BlockSpec((B3, EMBED_D), lambda t: (0, 0)),
        out_shape=jax.ShapeDtypeStruct((B3, EMBED_D), jnp.float32),
        scratch_shapes=[
            pltpu.VMEM((B3, EMBED_D), jnp.float32),
            pltpu.VMEM((B3, EMBED_D), jnp.float32),
        ],
        compiler_params=pltpu.CompilerParams(
            vmem_limit_bytes=100 * 1024 * 1024),
    )(emb2d, wih_t, whh_t, bias)


def kernel(triple_batch, triple_index, word_embed, p_content, W_ih, W_hh,
           b_ih, b_hh):
    flat_ids = jnp.transpose(triple_batch.astype(jnp.int32)).reshape(B3)
    p_content_pad = jnp.pad(
        p_content.astype(jnp.int32), ((0, 0), (0, 128 - C_LEN)))
    emb = _sc_gather(flat_ids, p_content_pad, word_embed)

    gate_scale = jnp.concatenate([
        jnp.full((2 * EMBED_D,), 0.5, jnp.float32),
        jnp.ones((EMBED_D,), jnp.float32),
        jnp.full((EMBED_D,), 0.5, jnp.float32),
    ])
    wih_t = (jnp.transpose(W_ih) * gate_scale[None, :]).astype(jnp.bfloat16)
    whh_t = (jnp.transpose(W_hh) * gate_scale[None, :]).astype(jnp.bfloat16)
    bias = jnp.broadcast_to(
        ((b_ih + b_hh) * gate_scale)[None, :], (8, 4 * EMBED_D))
    out = _lstm(emb.reshape(B3, C_LEN * EMBED_D), wih_t, whh_t, bias)
    return (out[0:1024], out[1024:2048], out[2048:3072])
